# Initial kernel scaffold; baseline (speedup 1.0000x reference)
import jax
import jax.numpy as jnp
from jax.experimental import pallas as pl


def kernel(x, edge_index, batch, W1, b1, W2, b2, W3, b3, W4, b4, Wc, bc):
    raise NotImplementedError("write your pallas kernel here")


# speedup vs baseline: 5.2633x; 5.2633x over previous
import functools

import jax
import jax.numpy as jnp
from jax import lax
from jax.experimental import pallas as pl
from jax.experimental.pallas import tpu as pltpu
from jax.experimental.pallas import tpu_sc as plsc

_N = 10000
_D = 128
_H = 128
_G = 32

_NTILES = 32
_CHUNK = 128
_NPAD = 10240
_ROWS_PER_TILE = _NPAD // 16
_RB = 2000
_GRID = _N // _RB


def _agg_body(t_hbm, rows_hbm, cols_hbm, out_hbm,
              ridx_v, cidx_v, gbuf_v, acc_sh, sem, n_chunks):
    c = lax.axis_index("c")
    s = lax.axis_index("s")

    def _zrow(i, carry):
        for j in range(_H // 16):
            gbuf_v[i, pl.ds(j * 16, 16)] = jnp.zeros((16,), jnp.float32)
        return carry
    lax.fori_loop(0, _CHUNK, _zrow, 0)
    for k in range(_ROWS_PER_TILE // _CHUNK):
        pltpu.sync_copy(gbuf_v, acc_sh.at[pl.ds(s * _ROWS_PER_TILE + k * _CHUNK, _CHUNK)])
    plsc.subcore_barrier()

    per_tile = n_chunks * _CHUNK
    base = (c * 16 + s) * per_tile

    def _chunk(i, carry):
        off = pl.multiple_of(base + i * _CHUNK, 8)
        pltpu.sync_copy(rows_hbm.at[pl.ds(off, _CHUNK)], ridx_v)
        pltpu.sync_copy(cols_hbm.at[pl.ds(off, _CHUNK)], cidx_v)
        pltpu.async_copy(t_hbm.at[ridx_v], gbuf_v, sem).wait()
        pltpu.sync_copy(gbuf_v, acc_sh.at[cidx_v], add=True)
        return carry
    lax.fori_loop(0, n_chunks, _chunk, 0)

    plsc.subcore_barrier()
    row0 = s * _ROWS_PER_TILE
    pltpu.sync_copy(acc_sh.at[pl.ds(row0, _ROWS_PER_TILE)],
                    out_hbm.at[c, pl.ds(row0, _ROWS_PER_TILE)])


def _make_agg(n_chunks):
    return pl.kernel(
        functools.partial(_agg_body, n_chunks=n_chunks),
        out_type=jax.ShapeDtypeStruct((2, _NPAD, _H), jnp.float32),
        mesh=plsc.VectorSubcoreMesh(core_axis_name="c", subcore_axis_name="s"),
        scratch_types=[
            pltpu.VMEM((_CHUNK,), jnp.int32),
            pltpu.VMEM((_CHUNK,), jnp.int32),
            pltpu.VMEM((_CHUNK, _H), jnp.float32),
            pltpu.VMEM_SHARED((_NPAD, _H), jnp.float32),
            pltpu.SemaphoreType.DMA,
        ],
    )


def _dotT(a, w):
    return lax.dot_general(a, w, (((1,), (1,)), ((), ())),
                           preferred_element_type=jnp.float32)


def _lin_body(x_ref, w_ref, b_ref, o_ref):
    o_ref[...] = _dotT(x_ref[...], w_ref[...]) + b_ref[...]


def _mid_body(p0_ref, p1_ref, t_ref, w_ref, b_ref, o_ref):
    r = jnp.maximum(p0_ref[0] + p1_ref[0] + t_ref[...], 0.0)
    o_ref[...] = _dotT(r, w_ref[...]) + b_ref[...]


def _head_body(p0_ref, p1_ref, t_ref, batch_ref, wc_ref, bc_ref, o_ref,
               sums_ref, cnts_ref):
    i = pl.program_id(0)

    @pl.when(i == 0)
    def _():
        sums_ref[...] = jnp.zeros_like(sums_ref)
        cnts_ref[...] = jnp.zeros_like(cnts_ref)

    h = p0_ref[0] + p1_ref[0] + t_ref[...]
    b = batch_ref[0, 0, :]
    gids = lax.broadcasted_iota(jnp.int32, (_G, _RB), 0)
    mask = (gids == b[None, :]).astype(jnp.float32)
    sums_ref[...] += lax.dot_general(mask, h, (((1,), (0,)), ((), ())),
                                     preferred_element_type=jnp.float32)
    cnts_ref[...] += jnp.sum(mask, axis=1, keepdims=True)

    @pl.when(i == _GRID - 1)
    def _():
        pooled = sums_ref[...] / jnp.maximum(cnts_ref[...], 1.0)
        logits = jnp.sum(pooled * wc_ref[...], axis=1, keepdims=True) + bc_ref[0, 0]
        o_ref[...] = 1.0 / (1.0 + jnp.exp(-logits))


_row_spec = pl.BlockSpec((_RB, _H), lambda i: (i, 0))
_p0_spec = pl.BlockSpec((1, _RB, _H), lambda i: (0, i, 0))
_p1_spec = pl.BlockSpec((1, _RB, _H), lambda i: (1, i, 0))
_w_spec = pl.BlockSpec((_H, _H), lambda i: (0, 0))
_b_spec = pl.BlockSpec((1, _H), lambda i: (0, 0))


def _lin(x, w, b):
    return pl.pallas_call(
        _lin_body,
        grid=(_GRID,),
        in_specs=[_row_spec, _w_spec, _b_spec],
        out_specs=_row_spec,
        out_shape=jax.ShapeDtypeStruct((_N, _H), jnp.float32),
    )(x, w, b.reshape(1, _H))


def _mid(parts, t, w, b):
    return pl.pallas_call(
        _mid_body,
        grid=(_GRID,),
        in_specs=[_p0_spec, _p1_spec, _row_spec, _w_spec, _b_spec],
        out_specs=_row_spec,
        out_shape=jax.ShapeDtypeStruct((_N, _H), jnp.float32),
    )(parts, parts, t, w, b.reshape(1, _H))


def _head(parts, t, batch3, wc, bc):
    return pl.pallas_call(
        _head_body,
        grid=(_GRID,),
        in_specs=[
            _p0_spec, _p1_spec, _row_spec,
            pl.BlockSpec((1, 1, _RB), lambda i: (i, 0, 0)),
            pl.BlockSpec((1, _H), lambda i: (0, 0)),
            pl.BlockSpec((1, 1), lambda i: (0, 0)),
        ],
        out_specs=pl.BlockSpec((_G, 1), lambda i: (0, 0)),
        out_shape=jax.ShapeDtypeStruct((_G, 1), jnp.float32),
        scratch_shapes=[
            pltpu.VMEM((_G, _H), jnp.float32),
            pltpu.VMEM((_G, 1), jnp.float32),
        ],
    )(parts, parts, t, batch3, wc, bc.reshape(1, 1))


def kernel(x, edge_index, batch, W1, b1, W2, b2, W3, b3, W4, b4, Wc, bc):
    e = edge_index.shape[1]
    per_tile_chunks = -(-e // (_NTILES * _CHUNK))
    e_pad = per_tile_chunks * _NTILES * _CHUNK
    pad = e_pad - e
    rows = jnp.concatenate([edge_index[0], jnp.zeros((pad,), jnp.int32)])
    cols = jnp.concatenate([edge_index[1], jnp.full((pad,), _N, jnp.int32)])
    batch3 = batch.reshape(_GRID, 1, _RB)

    agg = _make_agg(per_tile_chunks)

    t1 = _lin(x, W1, b1)
    p1 = agg(t1, rows, cols)
    t2 = _mid(p1, t1, W2, b2)
    p2 = agg(t2, rows, cols)
    t3 = _mid(p2, t2, W3, b3)
    p3 = agg(t3, rows, cols)
    t4 = _mid(p3, t3, W4, b4)
    p4 = agg(t4, rows, cols)
    return _head(p4, t4, batch3, Wc, bc)
